# baseline (speedup 1.0000x reference)
import functools

import jax
import jax.numpy as jnp
from jax import lax
from jax.experimental import pallas as pl
from jax.experimental.pallas import tpu as pltpu
from jax.experimental.pallas import tpu_sc as plsc

EMBED_DIM = 64
NC = 2
NS = 16
NW = NC * NS
CHUNK = 128
NBUF = 8
LANES = 16
NCH_GROUPS = CHUNK // LANES


def _gather_call(batch: int, fields: int):
    ntot = batch * fields
    per_w = ntot // NW
    b_per_w = batch // NW
    nch = per_w // CHUNK
    nouter = nch // NBUF
    ngrp = per_w // LANES
    mesh = plsc.VectorSubcoreMesh(core_axis_name="c", subcore_axis_name="s")

    @functools.partial(
        pl.kernel,
        mesh=mesh,
        out_type=jax.ShapeDtypeStruct((ntot, EMBED_DIM), jnp.float32),
        scratch_types=[
            pltpu.VMEM((fields, b_per_w), jnp.int32),
            pltpu.VMEM((nch, CHUNK), jnp.int32),
            pltpu.VMEM((ngrp, LANES), jnp.int32),
            pltpu.VMEM((ngrp, LANES), jnp.int32),
            [pltpu.VMEM((CHUNK, EMBED_DIM), jnp.float32) for _ in range(NBUF)],
            [pltpu.SemaphoreType.DMA for _ in range(NBUF)],
            [pltpu.SemaphoreType.DMA for _ in range(NBUF)],
        ],
        compiler_params=pltpu.CompilerParams(
            use_tc_tiling_on_sc=False, needs_layout_passes=False),
    )
    def k(idx_hbm, fvec_hbm, bvec_hbm, table_hbm, out_hbm,
          slab, pidx, fvs, bvs, rows, gsem, wsem):
        wid = lax.axis_index("s") * NC + lax.axis_index("c")
        base = wid * per_w
        pltpu.sync_copy(idx_hbm.at[:, pl.ds(wid * b_per_w, b_per_w)], slab)
        pltpu.sync_copy(fvec_hbm, fvs)
        pltpu.sync_copy(bvec_hbm, bvs)

        def trans_body(jo, _):
            for r in range(NCH_GROUPS):
                g = jo * NCH_GROUPS + r
                f = fvs[g, :]
                b = bvs[g, :]
                vals = plsc.load_gather(slab, [f, b])
                pidx[jo, pl.ds(r * LANES, LANES)] = vals
            return 0

        lax.fori_loop(0, nch, trans_body, 0)

        def gather(b, j):
            return pltpu.make_async_copy(
                table_hbm.at[pidx.at[j]], rows[b], gsem[b])

        def writeback(b, j):
            return pltpu.make_async_copy(
                rows[b], out_hbm.at[pl.ds(base + j * CHUNK, CHUNK)], wsem[b])

        for b in range(NBUF):
            gather(b, b).start()

        def outer(jo, _):
            for b in range(NBUF):
                j = jo * NBUF + b
                gather(b, j).wait()
                writeback(b, j).start()
                writeback(b, j).wait()
                gather(b, j + NBUF).start()
            return 0

        lax.fori_loop(0, nouter - 1, outer, 0)

        for b in range(NBUF):
            j = (nouter - 1) * NBUF + b
            gather(b, j).wait()
            writeback(b, j).start()
        for b in range(NBUF):
            j = (nouter - 1) * NBUF + b
            writeback(b, j).wait()

    return k


def kernel(nodes, table):
    batch, fields = nodes.shape
    per_w = batch * fields // NW
    nodes_t = nodes.T
    p = jnp.arange(per_w, dtype=jnp.int32)
    fvec = (p % fields).reshape(per_w // LANES, LANES)
    bvec = (p // fields).reshape(per_w // LANES, LANES)
    out = _gather_call(batch, fields)(nodes_t, fvec, bvec, table)
    return out.reshape(batch, fields, EMBED_DIM)


# speedup vs baseline: 1.1063x; 1.1063x over previous
import functools

import jax
import jax.numpy as jnp
from jax import lax
from jax.experimental import pallas as pl
from jax.experimental.pallas import tpu as pltpu
from jax.experimental.pallas import tpu_sc as plsc

EMBED_DIM = 64
ROWPAD = 128
NC = 2
NS = 16
NW = NC * NS
CHUNK = 128
NBUF = 4
LANES = 16
NCH_GROUPS = CHUNK // LANES
TBLK = 2048


def _table_widen(table_t):
    d, vocab = table_t.shape
    grid = (vocab + TBLK - 1) // TBLK

    def body(in_ref, o_ref):
        o_ref[...] = jnp.concatenate(
            [in_ref[...].T, jnp.zeros((TBLK, ROWPAD - d), jnp.float32)],
            axis=1)

    return pl.pallas_call(
        body,
        grid=(grid,),
        in_specs=[pl.BlockSpec((d, TBLK), lambda i: (0, i))],
        out_specs=pl.BlockSpec((TBLK, ROWPAD), lambda i: (i, 0)),
        out_shape=jax.ShapeDtypeStruct((vocab, ROWPAD), jnp.float32),
    )(table_t)


def _gather_call(batch: int, fields: int):
    ntot = batch * fields
    per_w = ntot // NW
    b_per_w = batch // NW
    nch = per_w // CHUNK
    nouter = nch // NBUF
    ngrp = per_w // LANES
    mesh = plsc.VectorSubcoreMesh(core_axis_name="c", subcore_axis_name="s")

    @functools.partial(
        pl.kernel,
        mesh=mesh,
        out_type=jax.ShapeDtypeStruct((ntot, EMBED_DIM), jnp.float32),
        scratch_types=[
            pltpu.VMEM((fields, b_per_w), jnp.int32),
            pltpu.VMEM((nch, CHUNK), jnp.int32),
            pltpu.VMEM((ngrp, LANES), jnp.int32),
            pltpu.VMEM((ngrp, LANES), jnp.int32),
            [pltpu.VMEM((CHUNK, ROWPAD), jnp.float32) for _ in range(NBUF)],
            [pltpu.SemaphoreType.DMA for _ in range(NBUF)],
            [pltpu.SemaphoreType.DMA for _ in range(NBUF)],
        ],
        compiler_params=pltpu.CompilerParams(
            use_tc_tiling_on_sc=False, needs_layout_passes=False),
    )
    def k(idx_hbm, fvec_hbm, bvec_hbm, table_hbm, out_hbm,
          slab, pidx, fvs, bvs, rows, gsem, wsem):
        wid = lax.axis_index("s") * NC + lax.axis_index("c")
        base = wid * per_w
        pltpu.sync_copy(idx_hbm.at[:, pl.ds(wid * b_per_w, b_per_w)], slab)
        pltpu.sync_copy(fvec_hbm, fvs)
        pltpu.sync_copy(bvec_hbm, bvs)

        def trans_body(jo, _):
            for r in range(NCH_GROUPS):
                g = jo * NCH_GROUPS + r
                f = fvs[g, :]
                b = bvs[g, :]
                vals = plsc.load_gather(slab, [f, b])
                pidx[jo, pl.ds(r * LANES, LANES)] = vals
            return 0

        lax.fori_loop(0, nch, trans_body, 0)

        def gather(b, j):
            return pltpu.make_async_copy(
                table_hbm.at[pidx.at[j]], rows[b], gsem[b])

        def writeback(b, j):
            return pltpu.make_async_copy(
                rows[b].at[:, pl.ds(0, EMBED_DIM)],
                out_hbm.at[pl.ds(base + j * CHUNK, CHUNK)], wsem[b])

        for b in range(NBUF):
            gather(b, b).start()

        def outer(jo, _):
            for b in range(NBUF):
                j = jo * NBUF + b
                gather(b, j).wait()
                writeback(b, j).start()
                writeback(b, j).wait()
                gather(b, j + NBUF).start()
            return 0

        lax.fori_loop(0, nouter - 1, outer, 0)

        for b in range(NBUF):
            j = (nouter - 1) * NBUF + b
            gather(b, j).wait()
            writeback(b, j).start()
        for b in range(NBUF):
            j = (nouter - 1) * NBUF + b
            writeback(b, j).wait()

    return k


def kernel(nodes, table):
    batch, fields = nodes.shape
    per_w = batch * fields // NW
    nodes_t = nodes.T
    table_w = _table_widen(table.T)
    p = jnp.arange(per_w, dtype=jnp.int32)
    fvec = (p % fields).reshape(per_w // LANES, LANES)
    bvec = (p // fields).reshape(per_w // LANES, LANES)
    out = _gather_call(batch, fields)(nodes_t, fvec, bvec, table_w)
    return out.reshape(batch, fields, EMBED_DIM)
